# Initial kernel scaffold; baseline (speedup 1.0000x reference)
import jax
import jax.numpy as jnp
from jax.experimental import pallas as pl


def kernel(input_ids, token_embedding, position_embedding, position_embedding_res):
    raise NotImplementedError("write your pallas kernel here")


# speedup vs baseline: 1.5182x; 1.5182x over previous
import functools

import jax
import jax.numpy as jnp
from jax import lax
from jax.experimental import pallas as pl
from jax.experimental.pallas import tpu as pltpu
from jax.experimental.pallas import tpu_sc as plsc

B, L, D = 4096, 200, 128
NCHUNK, CW = 5, 40
NC, NS = 2, 16
NW = NC * NS
RPW = B // NW

_mesh = plsc.VectorSubcoreMesh(core_axis_name="c", subcore_axis_name="s")


@functools.partial(
    pl.kernel,
    out_type=jax.ShapeDtypeStruct((B, L, D), jnp.float32),
    mesh=_mesh,
    scratch_types=[
        pltpu.VMEM((4, NCHUNK, CW), jnp.int32),
        pltpu.VMEM((L, D), jnp.float32),
        pltpu.VMEM((24, D), jnp.float32),
        pltpu.VMEM((2, L, D), jnp.float32),
        pltpu.SemaphoreType.DMA,
        pltpu.SemaphoreType.DMA,
        pltpu.SemaphoreType.DMA,
        pltpu.SemaphoreType.DMA,
        pltpu.SemaphoreType.DMA,
        pltpu.SemaphoreType.DMA,
    ],
)
def _embed(ids_hbm, pos_hbm, posres_hbm, table_hbm, out_hbm,
           idx_v, tmpl_v, p24_v, dest_v, g0, g1, o0, o1, i0, i1):
    gsem = (g0, g1)
    osem = (o0, o1)
    isem = (i0, i1)
    wid = lax.axis_index("s") * NC + lax.axis_index("c")
    row0 = wid * RPW

    pltpu.sync_copy(posres_hbm.at[pl.ds(0, L)], tmpl_v)
    pltpu.sync_copy(pos_hbm.at[pl.ds(0, 24)], p24_v)
    for j in range(20):
        for k in range(D // 16):
            tmpl_v[j, pl.ds(16 * k, 16)] = p24_v[j, pl.ds(16 * k, 16)]

    def idx_load(r, p):
        pltpu.async_copy(ids_hbm.at[row0 + r], idx_v.at[r & 3], isem[p])

    def wait_idx(r, p):
        pltpu.make_async_copy(ids_hbm.at[0], idx_v.at[r & 3], isem[p]).wait()

    def gathers(i, b):
        slot = i & 3
        for j in range(NCHUNK):
            pltpu.async_copy(
                table_hbm.at[idx_v.at[slot, j]],
                dest_v.at[b, pl.ds(j * CW, CW)],
                gsem[b],
            )

    def wait_g(b):
        pltpu.make_async_copy(out_hbm.at[0], dest_v.at[b], gsem[b]).wait()

    def add_tmpl(b):
        def row_body(g, carry):
            for r in (0, 1):
                j = 2 * g + r
                for k in range(D // 16):
                    t = tmpl_v[j, pl.ds(16 * k, 16)]
                    plsc.addupdate(dest_v.at[b, j, pl.ds(16 * k, 16)], t)
            return carry

        lax.fori_loop(0, L // 2, row_body, 0)

    def put_out(i, b):
        pltpu.async_copy(dest_v.at[b], out_hbm.at[row0 + i], osem[b])

    def wait_o(b):
        pltpu.make_async_copy(dest_v.at[b], out_hbm.at[0], osem[b]).wait()

    pltpu.sync_copy(ids_hbm.at[row0], idx_v.at[0])
    gathers(0, 0)
    idx_load(1, 1)

    def outer(g, carry):
        for b in (0, 1):
            i = 2 * g + b
            nb = 1 - b
            nxt = i + 1

            @pl.when(nxt < RPW)
            def _():
                @pl.when(nxt >= 2)
                def _():
                    wait_o(nb)

                wait_idx(nxt, nb)
                gathers(nxt, nb)

                @pl.when(nxt + 1 < RPW)
                def _():
                    idx_load(nxt + 1, b)

            wait_g(b)
            add_tmpl(b)
            put_out(i, b)
        return carry

    lax.fori_loop(0, RPW // 2, outer, 0)
    wait_o(0)
    wait_o(1)


def kernel(input_ids, token_embedding, position_embedding, position_embedding_res):
    ids = input_ids.astype(jnp.int32).reshape(B, NCHUNK, CW)
    return _embed(ids, position_embedding, position_embedding_res, token_embedding)


# speedup vs baseline: 1.7302x; 1.1396x over previous
import functools

import jax
import jax.numpy as jnp
from jax import lax
from jax.experimental import pallas as pl
from jax.experimental.pallas import tpu as pltpu
from jax.experimental.pallas import tpu_sc as plsc

B, L, D = 4096, 200, 128
NCHUNK, CW = 5, 40
NC, NS = 2, 16
NW = NC * NS
RPW = B // NW

_mesh = plsc.VectorSubcoreMesh(core_axis_name="c", subcore_axis_name="s")


@functools.partial(
    pl.kernel,
    out_type=jax.ShapeDtypeStruct((B, L, D), jnp.float32),
    mesh=_mesh,
    scratch_types=[
        pltpu.VMEM((4, NCHUNK, CW), jnp.int32),
        pltpu.VMEM((2, L, D), jnp.float32),
        pltpu.VMEM_SHARED((L, D), jnp.float32),
        pltpu.SemaphoreType.DMA,
        pltpu.SemaphoreType.DMA,
        pltpu.SemaphoreType.DMA,
        pltpu.SemaphoreType.DMA,
        pltpu.SemaphoreType.DMA,
        pltpu.SemaphoreType.DMA,
        pltpu.SemaphoreType.DMA,
        pltpu.SemaphoreType.DMA,
    ],
)
def _embed(ids_hbm, pos_hbm, posres_hbm, table_hbm, out_hbm,
           idx_v, dest_v, tmpl_s, g0, g1, o0, o1, i0, i1, f0, f1):
    gsem = (g0, g1)
    osem = (o0, o1)
    isem = (i0, i1)
    fsem = (f0, f1)
    cid = lax.axis_index("c")
    sid = lax.axis_index("s")
    wid = sid * NC + cid
    row0 = wid * RPW

    @pl.when(sid == 0)
    def _():
        pltpu.sync_copy(posres_hbm.at[pl.ds(0, L)], dest_v.at[0])
        pltpu.sync_copy(pos_hbm.at[pl.ds(0, 24)], dest_v.at[1, pl.ds(0, 24)])
        for j in range(20):
            for k in range(D // 16):
                dest_v[0, j, pl.ds(16 * k, 16)] = dest_v[1, j, pl.ds(16 * k, 16)]
        pltpu.sync_copy(dest_v.at[0], tmpl_s)

    plsc.subcore_barrier()

    def idx_load(r, p):
        pltpu.async_copy(ids_hbm.at[row0 + r], idx_v.at[r & 3], isem[p])

    def wait_idx(r, p):
        pltpu.make_async_copy(ids_hbm.at[0], idx_v.at[r & 3], isem[p]).wait()

    def fill(b):
        pltpu.async_copy(tmpl_s, dest_v.at[b], fsem[b])

    def wait_f(b):
        pltpu.make_async_copy(tmpl_s, dest_v.at[b], fsem[b]).wait()

    def gathers(i, b):
        slot = i & 3
        for j in range(NCHUNK):
            pltpu.async_copy(
                table_hbm.at[idx_v.at[slot, j]],
                dest_v.at[b, pl.ds(j * CW, CW)],
                gsem[b],
                add=True,
            )

    def wait_g(b):
        pltpu.make_async_copy(out_hbm.at[0], dest_v.at[b], gsem[b]).wait()

    def put_out(i, b):
        pltpu.async_copy(dest_v.at[b], out_hbm.at[row0 + i], osem[b])

    def wait_o(b):
        pltpu.make_async_copy(dest_v.at[b], out_hbm.at[0], osem[b]).wait()

    pltpu.sync_copy(ids_hbm.at[row0], idx_v.at[0])
    fill(0)
    wait_f(0)
    gathers(0, 0)
    idx_load(1, 1)

    def outer(g, carry):
        for b in (0, 1):
            i = 2 * g + b
            nb = 1 - b
            nxt = i + 1

            @pl.when(nxt < RPW)
            def _():
                @pl.when(nxt >= 2)
                def _():
                    wait_o(nb)

                fill(nb)
                wait_idx(nxt, nb)
                wait_f(nb)
                gathers(nxt, nb)

                @pl.when(nxt + 1 < RPW)
                def _():
                    idx_load(nxt + 1, b)

            wait_g(b)
            put_out(i, b)
        return carry

    lax.fori_loop(0, RPW // 2, outer, 0)
    wait_o(0)
    wait_o(1)


def kernel(input_ids, token_embedding, position_embedding, position_embedding_res):
    ids = input_ids.astype(jnp.int32).reshape(B, NCHUNK, CW)
    return _embed(ids, position_embedding, position_embedding_res, token_embedding)


# speedup vs baseline: 1.7303x; 1.0001x over previous
import functools

import jax
import jax.numpy as jnp
from jax import lax
from jax.experimental import pallas as pl
from jax.experimental.pallas import tpu as pltpu
from jax.experimental.pallas import tpu_sc as plsc

B, L, D = 4096, 200, 128
CHUNKS = ((0, 104), (104, 96))
NC, NS = 2, 16
NW = NC * NS
RPW = B // NW

_mesh = plsc.VectorSubcoreMesh(core_axis_name="c", subcore_axis_name="s")


@functools.partial(
    pl.kernel,
    out_type=jax.ShapeDtypeStruct((B, L, D), jnp.float32),
    mesh=_mesh,
    scratch_types=[
        pltpu.VMEM((4 * L,), jnp.int32),
        pltpu.VMEM((2, L, D), jnp.float32),
        pltpu.VMEM_SHARED((L, D), jnp.float32),
        pltpu.SemaphoreType.DMA,
        pltpu.SemaphoreType.DMA,
        pltpu.SemaphoreType.DMA,
        pltpu.SemaphoreType.DMA,
        pltpu.SemaphoreType.DMA,
        pltpu.SemaphoreType.DMA,
        pltpu.SemaphoreType.DMA,
        pltpu.SemaphoreType.DMA,
    ],
)
def _embed(ids_hbm, pos_hbm, posres_hbm, table_hbm, out_hbm,
           idx_v, dest_v, tmpl_s, g0, g1, o0, o1, i0, i1, f0, f1):
    gsem = (g0, g1)
    osem = (o0, o1)
    isem = (i0, i1)
    fsem = (f0, f1)
    cid = lax.axis_index("c")
    sid = lax.axis_index("s")
    wid = sid * NC + cid
    row0 = wid * RPW

    @pl.when(sid == 0)
    def _():
        pltpu.sync_copy(posres_hbm.at[pl.ds(0, L)], dest_v.at[0])
        pltpu.sync_copy(pos_hbm.at[pl.ds(0, 24)], dest_v.at[1, pl.ds(0, 24)])
        for j in range(20):
            for k in range(D // 16):
                dest_v[0, j, pl.ds(16 * k, 16)] = dest_v[1, j, pl.ds(16 * k, 16)]
        pltpu.sync_copy(dest_v.at[0], tmpl_s)

    plsc.subcore_barrier()

    def idx_load(r, p):
        pltpu.async_copy(
            ids_hbm.at[pl.ds((row0 + r) * L, L)],
            idx_v.at[pl.ds((r & 3) * L, L)],
            isem[p],
        )

    def wait_idx(r, p):
        pltpu.make_async_copy(
            ids_hbm.at[pl.ds(0, L)], idx_v.at[pl.ds(0, L)], isem[p]
        ).wait()

    def fill(b):
        pltpu.async_copy(tmpl_s, dest_v.at[b], fsem[b])

    def wait_f(b):
        pltpu.make_async_copy(tmpl_s, dest_v.at[b], fsem[b]).wait()

    def gathers(i, b):
        base = (i & 3) * L
        for off, ln in CHUNKS:
            pltpu.async_copy(
                table_hbm.at[idx_v.at[pl.ds(base + off, ln)]],
                dest_v.at[b, pl.ds(off, ln)],
                gsem[b],
                add=True,
            )

    def wait_g(b):
        pltpu.make_async_copy(out_hbm.at[0], dest_v.at[b], gsem[b]).wait()

    def put_out(i, b):
        pltpu.async_copy(dest_v.at[b], out_hbm.at[row0 + i], osem[b])

    def wait_o(b):
        pltpu.make_async_copy(dest_v.at[b], out_hbm.at[0], osem[b]).wait()

    pltpu.sync_copy(ids_hbm.at[pl.ds(row0 * L, L)], idx_v.at[pl.ds(0, L)])
    fill(0)
    wait_f(0)
    gathers(0, 0)
    idx_load(1, 1)

    def outer(g, carry):
        for b in (0, 1):
            i = 2 * g + b
            nb = 1 - b
            nxt = i + 1

            @pl.when(nxt < RPW)
            def _():
                @pl.when(nxt >= 2)
                def _():
                    wait_o(nb)

                fill(nb)
                wait_idx(nxt, nb)
                wait_f(nb)
                gathers(nxt, nb)

                @pl.when(nxt + 1 < RPW)
                def _():
                    idx_load(nxt + 1, b)

            wait_g(b)
            put_out(i, b)
        return carry

    lax.fori_loop(0, RPW // 2, outer, 0)
    wait_o(0)
    wait_o(1)


def kernel(input_ids, token_embedding, position_embedding, position_embedding_res):
    ids = input_ids.astype(jnp.int32).reshape(B * L)
    return _embed(ids, position_embedding, position_embedding_res, token_embedding)


# speedup vs baseline: 1.8198x; 1.0517x over previous
import functools

import jax
import jax.numpy as jnp
from jax import lax
from jax.experimental import pallas as pl
from jax.experimental.pallas import tpu as pltpu
from jax.experimental.pallas import tpu_sc as plsc

B, L, D = 4096, 200, 128
CHUNKS = ((0, 104), (104, 96))
NBUF = 3
NC, NS = 2, 16
NW = NC * NS
RPW = B // NW

_mesh = plsc.VectorSubcoreMesh(core_axis_name="c", subcore_axis_name="s")


@functools.partial(
    pl.kernel,
    out_type=jax.ShapeDtypeStruct((B, L, D), jnp.float32),
    mesh=_mesh,
    scratch_types=[
        pltpu.VMEM((4 * L,), jnp.int32),
        pltpu.VMEM((L, D), jnp.float32),
        pltpu.VMEM((NBUF, L, D), jnp.float32),
        pltpu.SemaphoreType.DMA,
        pltpu.SemaphoreType.DMA,
        pltpu.SemaphoreType.DMA,
        pltpu.SemaphoreType.DMA,
        pltpu.SemaphoreType.DMA,
        pltpu.SemaphoreType.DMA,
        pltpu.SemaphoreType.DMA,
        pltpu.SemaphoreType.DMA,
        pltpu.SemaphoreType.DMA,
    ],
)
def _embed(ids_hbm, pos_hbm, posres_hbm, table_hbm, out_hbm,
           idx_v, tmpl_v, dest_v, g0, g1, g2, o0, o1, o2, i0, i1, i2):
    gsem = (g0, g1, g2)
    osem = (o0, o1, o2)
    isem = (i0, i1, i2)
    wid = lax.axis_index("s") * NC + lax.axis_index("c")
    row0 = wid * RPW

    pltpu.sync_copy(posres_hbm.at[pl.ds(0, L)], tmpl_v)
    pltpu.sync_copy(pos_hbm.at[pl.ds(0, 24)], dest_v.at[0, pl.ds(0, 24)])
    for j in range(20):
        for k in range(D // 16):
            tmpl_v[j, pl.ds(16 * k, 16)] = dest_v[0, j, pl.ds(16 * k, 16)]

    def idx_load(r, p):
        pltpu.async_copy(
            ids_hbm.at[pl.ds((row0 + r) * L, L)],
            idx_v.at[pl.ds((r & 3) * L, L)],
            isem[p],
        )

    def wait_idx(r, p):
        pltpu.make_async_copy(
            ids_hbm.at[pl.ds(0, L)], idx_v.at[pl.ds(0, L)], isem[p]
        ).wait()

    def gathers(i, b):
        base = (i & 3) * L
        for off, ln in CHUNKS:
            pltpu.async_copy(
                table_hbm.at[idx_v.at[pl.ds(base + off, ln)]],
                dest_v.at[b, pl.ds(off, ln)],
                gsem[b],
            )

    def wait_g(b):
        pltpu.make_async_copy(out_hbm.at[0], dest_v.at[b], gsem[b]).wait()

    def add_tmpl(b):
        def row_body(g, carry):
            for r in (0, 1):
                j = 2 * g + r
                for k in range(D // 16):
                    t = tmpl_v[j, pl.ds(16 * k, 16)]
                    plsc.addupdate(dest_v.at[b, j, pl.ds(16 * k, 16)], t)
            return carry

        lax.fori_loop(0, L // 2, row_body, 0)

    def put_out(i, b):
        pltpu.async_copy(dest_v.at[b], out_hbm.at[row0 + i], osem[b])

    def wait_o(b):
        pltpu.make_async_copy(dest_v.at[b], out_hbm.at[0], osem[b]).wait()

    pltpu.sync_copy(ids_hbm.at[pl.ds(row0 * L, L)], idx_v.at[pl.ds(0, L)])
    gathers(0, 0)
    idx_load(1, 1)

    def outer(g, carry):
        for r in range(NBUF):
            i = NBUF * g + r
            nb = (r + 1) % NBUF
            nxt = i + 1

            @pl.when(nxt < RPW)
            def _():
                @pl.when(nxt >= NBUF)
                def _():
                    wait_o(nb)

                wait_idx(nxt, nb)
                gathers(nxt, nb)

                @pl.when(nxt + 1 < RPW)
                def _():
                    idx_load(nxt + 1, (r + 2) % NBUF)

            @pl.when(i < RPW)
            def _():
                wait_g(r)
                add_tmpl(r)
                put_out(i, r)
        return carry

    lax.fori_loop(0, (RPW + NBUF - 1) // NBUF, outer, 0)
    wait_o(0)
    wait_o(1)
    wait_o(2)


def kernel(input_ids, token_embedding, position_embedding, position_embedding_res):
    ids = input_ids.astype(jnp.int32).reshape(B * L)
    return _embed(ids, position_embedding, position_embedding_res, token_embedding)
